# baseline (speedup 1.0000x reference)
import functools

import jax
import jax.numpy as jnp
from jax import lax
from jax.experimental import pallas as pl
from jax.experimental.pallas import tpu as pltpu
from jax.experimental.pallas import tpu_sc as plsc

_D = 128
_DH = 64
_NPAD = 10240
_BM = 1024
_NC = 2
_NS = 16
_KCH = 128
_NBUF = 6
_RPT = _NPAD // _NS


def _mesh():
    return plsc.VectorSubcoreMesh(core_axis_name="c", subcore_axis_name="s")


def _make_deg_kernel(nchunk):
    @functools.partial(
        pl.kernel,
        out_type=jax.ShapeDtypeStruct((_NC, _NPAD, 16), jnp.float32),
        mesh=_mesh(),
        compiler_params=pltpu.CompilerParams(use_tc_tiling_on_sc=False),
        scratch_types=[
            pltpu.VMEM((nchunk, _KCH), jnp.int32),
            pltpu.VMEM((_KCH, 16), jnp.float32),
            pltpu.VMEM_SHARED((_NPAD, 16), jnp.float32),
        ],
    )
    def deg_kernel(dst_hbm, out_hbm, dst_v, ones_v, acc):
        c = lax.axis_index("c")
        s = lax.axis_index("s")
        pltpu.sync_copy(dst_hbm.at[s], dst_v)

        def fill(i, _):
            ones_v[i, :] = jnp.full((16,), 1.0, jnp.float32)
            return ()

        lax.fori_loop(0, _KCH, fill, ())
        sl = pl.ds(s * _RPT, _RPT)
        for r in range(_RPT // _KCH):
            pltpu.sync_copy(ones_v, acc.at[pl.ds(s * _RPT + r * _KCH, _KCH)])
        plsc.subcore_barrier()

        def step(j, _):
            pltpu.sync_copy(ones_v, acc.at[dst_v.at[j]], add=True)
            return ()

        lax.fori_loop(0, nchunk, step, ())
        plsc.subcore_barrier()
        pltpu.sync_copy(acc.at[sl], out_hbm.at[c].at[sl])

    return deg_kernel


def _make_agg_kernel(nchunk):
    ntail = nchunk % _NBUF
    nloop = nchunk - ntail

    @functools.partial(
        pl.kernel,
        out_type=jax.ShapeDtypeStruct((_NC, _NPAD, _DH), jnp.float32),
        mesh=_mesh(),
        compiler_params=pltpu.CompilerParams(use_tc_tiling_on_sc=False),
        scratch_types=[
            pltpu.VMEM((nchunk, _KCH), jnp.int32),
            pltpu.VMEM((nchunk, _KCH), jnp.int32),
            pltpu.VMEM((_NBUF, _KCH, _DH), jnp.float32),
            [pltpu.SemaphoreType.DMA] * _NBUF,
            pltpu.VMEM_SHARED((_NPAD, _DH), jnp.float32),
        ],
    )
    def agg_kernel(zs_hbm, src_hbm, dst_hbm, out_hbm, src_v, dst_v, rows, sems, acc):
        c = lax.axis_index("c")
        s = lax.axis_index("s")
        zh = zs_hbm.at[c]
        pltpu.sync_copy(src_hbm.at[s], src_v)
        pltpu.sync_copy(dst_hbm.at[s], dst_v)
        sl = pl.ds(s * _RPT, _RPT)
        pltpu.sync_copy(zh.at[sl], acc.at[sl])
        plsc.subcore_barrier()

        for b in range(_NBUF):
            pltpu.async_copy(zh.at[src_v.at[b]], rows.at[b], sems[b])

        def group(g, _):
            j0 = g * _NBUF
            for b in range(_NBUF):
                j = j0 + b
                pltpu.make_async_copy(
                    zh.at[src_v.at[j]], rows.at[b], sems[b]
                ).wait()
                pltpu.sync_copy(rows.at[b], acc.at[dst_v.at[j]], add=True)
                nj = j + _NBUF

                @pl.when(nj < nchunk)
                def _():
                    pltpu.async_copy(zh.at[src_v.at[nj]], rows.at[b], sems[b])
            return ()

        lax.fori_loop(0, nloop // _NBUF, group, ())
        for j in range(nloop, nchunk):
            b = j % _NBUF
            pltpu.make_async_copy(
                zh.at[src_v.at[j]], rows.at[b], sems[b]
            ).wait()
            pltpu.sync_copy(rows.at[b], acc.at[dst_v.at[j]], add=True)
        plsc.subcore_barrier()
        pltpu.sync_copy(acc.at[sl], out_hbm.at[c].at[sl])

    return agg_kernel


def _prep_tc(xp, W, degc):
    def body(x_ref, w_ref, d_ref, z_ref, dis_ref):
        dis = lax.rsqrt(d_ref[...][:, 0:1])
        z = dis * jnp.dot(x_ref[...], w_ref[...],
                          preferred_element_type=jnp.float32)
        z_ref[0] = z[:, :_DH]
        z_ref[1] = z[:, _DH:]
        dis_ref[...] = jnp.broadcast_to(dis, (_BM, _D))

    return pl.pallas_call(
        body,
        grid=(_NPAD // _BM,),
        in_specs=[
            pl.BlockSpec((_BM, _D), lambda i: (i, 0)),
            pl.BlockSpec((_D, _D), lambda i: (0, 0)),
            pl.BlockSpec((_BM, 16), lambda i: (i, 0)),
        ],
        out_specs=[
            pl.BlockSpec((_NC, _BM, _DH), lambda i: (0, i, 0)),
            pl.BlockSpec((_BM, _D), lambda i: (i, 0)),
        ],
        out_shape=[
            jax.ShapeDtypeStruct((_NC, _NPAD, _DH), jnp.float32),
            jax.ShapeDtypeStruct((_NPAD, _D), jnp.float32),
        ],
    )(xp, W, degc)


def _mid_tc(ag, dis, b, W):
    def body(ag_ref, dis_ref, b_ref, w_ref, z_ref):
        agg = jnp.concatenate([ag_ref[0], ag_ref[1]], axis=1)
        h = jnp.maximum(dis_ref[...] * agg + b_ref[...], 0.0)
        z = dis_ref[...] * jnp.dot(h, w_ref[...],
                                   preferred_element_type=jnp.float32)
        z_ref[0] = z[:, :_DH]
        z_ref[1] = z[:, _DH:]

    return pl.pallas_call(
        body,
        grid=(_NPAD // _BM,),
        in_specs=[
            pl.BlockSpec((_NC, _BM, _DH), lambda i: (0, i, 0)),
            pl.BlockSpec((_BM, _D), lambda i: (i, 0)),
            pl.BlockSpec((1, _D), lambda i: (0, 0)),
            pl.BlockSpec((_D, _D), lambda i: (0, 0)),
        ],
        out_specs=pl.BlockSpec((_NC, _BM, _DH), lambda i: (0, i, 0)),
        out_shape=jax.ShapeDtypeStruct((_NC, _NPAD, _DH), jnp.float32),
    )(ag, dis, b, W)


def _final_tc(ag, dis, b, n):
    bm = 400
    def body(ag_ref, dis_ref, b_ref, o_ref):
        agg = jnp.concatenate([ag_ref[0], ag_ref[1]], axis=1)
        o_ref[...] = jnp.maximum(dis_ref[...] * agg + b_ref[...], 0.0)

    return pl.pallas_call(
        body,
        grid=(n // bm,),
        in_specs=[
            pl.BlockSpec((_NC, bm, _DH), lambda i: (0, i, 0)),
            pl.BlockSpec((bm, _D), lambda i: (i, 0)),
            pl.BlockSpec((1, _D), lambda i: (0, 0)),
        ],
        out_specs=pl.BlockSpec((bm, _D), lambda i: (i, 0)),
        out_shape=jax.ShapeDtypeStruct((n, _D), jnp.float32),
    )(ag, dis, b)


@jax.jit
def kernel(x, edge_index, W1, b1, W2, b2, W3, b3):
    n = x.shape[0]
    e = edge_index.shape[1]

    eblk = _NS * _KCH
    e_pad = ((e + eblk - 1) // eblk) * eblk
    nchunk = e_pad // eblk
    src = jnp.full((e_pad,), _NPAD - 1, jnp.int32).at[:e].set(edge_index[0])
    dst = jnp.full((e_pad,), _NPAD - 1, jnp.int32).at[:e].set(edge_index[1])
    src3 = src.reshape(_NS, nchunk, _KCH)
    dst3 = dst.reshape(_NS, nchunk, _KCH)

    deg_kernel = _make_deg_kernel(nchunk)
    agg_kernel = _make_agg_kernel(nchunk)

    dp = deg_kernel(dst3)
    degc = dp[0]

    z1, dis = _prep_tc(x, W1, degc)
    a1 = agg_kernel(z1, src3, dst3)
    z2 = _mid_tc(a1, dis, b1.reshape(1, _D), W2)
    a2 = agg_kernel(z2, src3, dst3)
    z3 = _mid_tc(a2, dis, b2.reshape(1, _D), W3)
    a3 = agg_kernel(z3, src3, dst3)
    return _final_tc(a3, dis, b3.reshape(1, _D), n)


# speedup vs baseline: 1.0106x; 1.0106x over previous
import functools

import jax
import jax.numpy as jnp
from jax import lax
from jax.experimental import pallas as pl
from jax.experimental.pallas import tpu as pltpu
from jax.experimental.pallas import tpu_sc as plsc

_D = 128
_DH = 64
_NPAD = 10240
_BM = 1024
_NC = 2
_NS = 16
_KCH = 128
_NBUF = 6
_RPT = _NPAD // _NS


def _mesh():
    return plsc.VectorSubcoreMesh(core_axis_name="c", subcore_axis_name="s")


def _make_deg_kernel(nchunk):
    @functools.partial(
        pl.kernel,
        out_type=jax.ShapeDtypeStruct((_NC, _NPAD, 16), jnp.float32),
        mesh=_mesh(),
        compiler_params=pltpu.CompilerParams(use_tc_tiling_on_sc=False),
        scratch_types=[
            pltpu.VMEM((nchunk, _KCH), jnp.int32),
            pltpu.VMEM((_KCH, 16), jnp.float32),
            pltpu.VMEM_SHARED((_NPAD, 16), jnp.float32),
        ],
    )
    def deg_kernel(dst_hbm, out_hbm, dst_v, ones_v, acc):
        c = lax.axis_index("c")
        s = lax.axis_index("s")
        pltpu.sync_copy(dst_hbm.at[s], dst_v)

        def fill(i, _):
            ones_v[i, :] = jnp.full((16,), 1.0, jnp.float32)
            return ()

        lax.fori_loop(0, _KCH, fill, ())
        sl = pl.ds(s * _RPT, _RPT)
        for r in range(_RPT // _KCH):
            pltpu.sync_copy(ones_v, acc.at[pl.ds(s * _RPT + r * _KCH, _KCH)])
        plsc.subcore_barrier()

        def step(j, _):
            pltpu.sync_copy(ones_v, acc.at[dst_v.at[j]], add=True)
            return ()

        lax.fori_loop(0, nchunk, step, ())
        plsc.subcore_barrier()
        pltpu.sync_copy(acc.at[sl], out_hbm.at[c].at[sl])

    return deg_kernel


def _make_agg_kernel(nchunk):
    ntail = nchunk % _NBUF
    nloop = nchunk - ntail

    @functools.partial(
        pl.kernel,
        out_type=jax.ShapeDtypeStruct((_NC, _NPAD, _DH), jnp.float32),
        mesh=_mesh(),
        compiler_params=pltpu.CompilerParams(use_tc_tiling_on_sc=False),
        scratch_types=[
            pltpu.VMEM((nchunk, _KCH), jnp.int32),
            pltpu.VMEM((nchunk, _KCH), jnp.int32),
            pltpu.VMEM((_NBUF, _KCH, _DH), jnp.float32),
            [pltpu.SemaphoreType.DMA] * _NBUF,
            pltpu.VMEM_SHARED((_NPAD, _DH), jnp.float32),
        ],
    )
    def agg_kernel(zs_hbm, src_hbm, dst_hbm, out_hbm, src_v, dst_v, rows, sems, acc):
        c = lax.axis_index("c")
        s = lax.axis_index("s")
        zh = zs_hbm.at[c]
        pltpu.sync_copy(src_hbm.at[s], src_v)
        pltpu.sync_copy(dst_hbm.at[s], dst_v)
        sl = pl.ds(s * _RPT, _RPT)
        pltpu.sync_copy(zh.at[sl], acc.at[sl])
        plsc.subcore_barrier()

        for b in range(_NBUF):
            pltpu.async_copy(zh.at[src_v.at[b]], rows.at[b], sems[b])

        def group(g, _):
            j0 = g * _NBUF
            for b in range(_NBUF):
                j = j0 + b
                pltpu.make_async_copy(
                    zh.at[src_v.at[j]], rows.at[b], sems[b]
                ).wait()
                pltpu.sync_copy(rows.at[b], acc.at[dst_v.at[j]], add=True)
                nj = j + _NBUF

                @pl.when(nj < nchunk)
                def _():
                    pltpu.async_copy(zh.at[src_v.at[nj]], rows.at[b], sems[b])
            return ()

        lax.fori_loop(0, nloop // _NBUF, group, ())
        for j in range(nloop, nchunk):
            b = j % _NBUF
            pltpu.make_async_copy(
                zh.at[src_v.at[j]], rows.at[b], sems[b]
            ).wait()
            pltpu.sync_copy(rows.at[b], acc.at[dst_v.at[j]], add=True)
        plsc.subcore_barrier()
        pltpu.sync_copy(acc.at[sl], out_hbm.at[c].at[sl])

    return agg_kernel


def _matmul_tc(xp, W):
    def body(x_ref, w_ref, o_ref):
        o_ref[...] = jnp.dot(x_ref[...], w_ref[...],
                             preferred_element_type=jnp.float32)

    return pl.pallas_call(
        body,
        grid=(_NPAD // _BM,),
        in_specs=[
            pl.BlockSpec((_BM, _D), lambda i: (i, 0)),
            pl.BlockSpec((_D, _D), lambda i: (0, 0)),
        ],
        out_specs=pl.BlockSpec((_BM, _D), lambda i: (i, 0)),
        out_shape=jax.ShapeDtypeStruct((_NPAD, _D), jnp.float32),
    )(xp, W)


def _scale_tc(z0, degc):
    def body(z0_ref, d_ref, z_ref, dis_ref):
        dis = lax.rsqrt(d_ref[...][:, 0:1])
        z = dis * z0_ref[...]
        z_ref[0] = z[:, :_DH]
        z_ref[1] = z[:, _DH:]
        dis_ref[...] = jnp.broadcast_to(dis, (_BM, _D))

    return pl.pallas_call(
        body,
        grid=(_NPAD // _BM,),
        in_specs=[
            pl.BlockSpec((_BM, _D), lambda i: (i, 0)),
            pl.BlockSpec((_BM, 16), lambda i: (i, 0)),
        ],
        out_specs=[
            pl.BlockSpec((_NC, _BM, _DH), lambda i: (0, i, 0)),
            pl.BlockSpec((_BM, _D), lambda i: (i, 0)),
        ],
        out_shape=[
            jax.ShapeDtypeStruct((_NC, _NPAD, _DH), jnp.float32),
            jax.ShapeDtypeStruct((_NPAD, _D), jnp.float32),
        ],
    )(z0, degc)


def _mid_tc(ag, dis, b, W):
    def body(ag_ref, dis_ref, b_ref, w_ref, z_ref):
        agg = jnp.concatenate([ag_ref[0], ag_ref[1]], axis=1)
        h = jnp.maximum(dis_ref[...] * agg + b_ref[...], 0.0)
        z = dis_ref[...] * jnp.dot(h, w_ref[...],
                                   preferred_element_type=jnp.float32)
        z_ref[0] = z[:, :_DH]
        z_ref[1] = z[:, _DH:]

    return pl.pallas_call(
        body,
        grid=(_NPAD // _BM,),
        in_specs=[
            pl.BlockSpec((_NC, _BM, _DH), lambda i: (0, i, 0)),
            pl.BlockSpec((_BM, _D), lambda i: (i, 0)),
            pl.BlockSpec((1, _D), lambda i: (0, 0)),
            pl.BlockSpec((_D, _D), lambda i: (0, 0)),
        ],
        out_specs=pl.BlockSpec((_NC, _BM, _DH), lambda i: (0, i, 0)),
        out_shape=jax.ShapeDtypeStruct((_NC, _NPAD, _DH), jnp.float32),
    )(ag, dis, b, W)


def _final_tc(ag, dis, b):
    def body(ag_ref, dis_ref, b_ref, o_ref):
        agg = jnp.concatenate([ag_ref[0], ag_ref[1]], axis=1)
        o_ref[...] = jnp.maximum(dis_ref[...] * agg + b_ref[...], 0.0)

    return pl.pallas_call(
        body,
        grid=(_NPAD // _BM,),
        in_specs=[
            pl.BlockSpec((_NC, _BM, _DH), lambda i: (0, i, 0)),
            pl.BlockSpec((_BM, _D), lambda i: (i, 0)),
            pl.BlockSpec((1, _D), lambda i: (0, 0)),
        ],
        out_specs=pl.BlockSpec((_BM, _D), lambda i: (i, 0)),
        out_shape=jax.ShapeDtypeStruct((_NPAD, _D), jnp.float32),
    )(ag, dis, b)


@jax.jit
def kernel(x, edge_index, W1, b1, W2, b2, W3, b3):
    n = x.shape[0]
    e = edge_index.shape[1]

    xp = jnp.zeros((_NPAD, _D), jnp.float32).at[:n].set(x)

    eblk = _NS * _KCH
    e_pad = ((e + eblk - 1) // eblk) * eblk
    nchunk = e_pad // eblk
    src = jnp.full((e_pad,), _NPAD - 1, jnp.int32).at[:e].set(edge_index[0])
    dst = jnp.full((e_pad,), _NPAD - 1, jnp.int32).at[:e].set(edge_index[1])
    src3 = src.reshape(_NS, nchunk, _KCH)
    dst3 = dst.reshape(_NS, nchunk, _KCH)

    deg_kernel = _make_deg_kernel(nchunk)
    agg_kernel = _make_agg_kernel(nchunk)

    dp = deg_kernel(dst3)
    z0 = _matmul_tc(xp, W1)
    degc = dp[0]
    z1, dis = _scale_tc(z0, degc)
    a1 = agg_kernel(z1, src3, dst3)
    z2 = _mid_tc(a1, dis, b1.reshape(1, _D), W2)
    a2 = agg_kernel(z2, src3, dst3)
    z3 = _mid_tc(a2, dis, b2.reshape(1, _D), W3)
    a3 = agg_kernel(z3, src3, dst3)
    h = _final_tc(a3, dis, b3.reshape(1, _D))
    return h[:n]
